# baseline (speedup 1.0000x reference)
import functools

import jax
import jax.numpy as jnp
from jax import lax
from jax.experimental import pallas as pl
from jax.experimental.pallas import tpu as pltpu
from jax.experimental.pallas import tpu_sc as plsc

B = 50000
D = 128
E = 128
C = 16
S1 = 11

NC, NS = 2, 16
NW = NC * NS
RPS = 8
IPS = RPS * S1
IDXW = 96
NSTEPS = 196
BPW = NSTEPS * RPS
BPAD = NW * BPW


def _sc_gather_sum(features, idx_grp):
    mesh = plsc.VectorSubcoreMesh(core_axis_name="c", subcore_axis_name="s")
    NBUF = 4
    NGRP = NSTEPS // NBUF

    @functools.partial(
        pl.kernel,
        mesh=mesh,
        out_type=jax.ShapeDtypeStruct((BPAD, D), jnp.float32),
        scratch_types=[
            pltpu.VMEM((NSTEPS, IDXW), jnp.int32),
            pltpu.VMEM((NBUF, IDXW, D), jnp.float32),
            pltpu.VMEM((NBUF, RPS, D), jnp.float32),
            pltpu.SemaphoreType.DMA((NBUF,)),
            pltpu.SemaphoreType.DMA((NBUF,)),
        ],
    )
    def k(idx_hbm, feat_hbm, out_hbm, idx_slab, rows, acc, gsem, osem):
        wid = lax.axis_index("s") * NC + lax.axis_index("c")
        pltpu.sync_copy(idx_hbm.at[wid], idx_slab)

        for b in range(NBUF):
            pltpu.async_copy(feat_hbm.at[idx_slab.at[b]], rows.at[b], gsem.at[b])

        def group(p, carry):
            for b in range(NBUF):
                i = p * NBUF + b
                pltpu.make_async_copy(
                    feat_hbm.at[idx_slab.at[i]], rows.at[b], gsem.at[b]).wait()
                @pl.when(p > 0)
                def _():
                    pltpu.make_async_copy(
                        acc.at[b], out_hbm.at[pl.ds(0, RPS)], osem.at[b]).wait()
                for r in range(RPS):
                    for g in range(D // 16):
                        v = rows[b, r * S1, pl.ds(g * 16, 16)]
                        for j in range(1, S1):
                            v = v + rows[b, r * S1 + j, pl.ds(g * 16, 16)]
                        acc[b, r, pl.ds(g * 16, 16)] = v
                pltpu.async_copy(
                    acc.at[b], out_hbm.at[pl.ds(wid * BPW + i * RPS, RPS)],
                    osem.at[b])
                @pl.when(i + NBUF < NSTEPS)
                def _():
                    pltpu.async_copy(
                        feat_hbm.at[idx_slab.at[i + NBUF]], rows.at[b],
                        gsem.at[b])
            return carry

        lax.fori_loop(0, NGRP, group, 0)
        for b in range(NBUF):
            pltpu.make_async_copy(
                acc.at[b], out_hbm.at[pl.ds(0, RPS)], osem.at[b]).wait()

    return k(idx_grp, features)


BLK = 1000


def _tc_head(sums, W0, W_cls):
    def body(x_ref, w0_ref, wc_ref, o_ref):
        x = x_ref[...] * (1.0 / S1)
        h = lax.dot_general(x, w0_ref[...], (((1,), (1,)), ((), ())),
                            preferred_element_type=jnp.float32)
        h = jnp.maximum(h, 0.0)
        s = lax.dot_general(h, wc_ref[...], (((1,), (1,)), ((), ())),
                            preferred_element_type=jnp.float32)
        o_ref[...] = jax.nn.sigmoid(s)

    return pl.pallas_call(
        body,
        grid=(B // BLK,),
        in_specs=[
            pl.BlockSpec((BLK, D), lambda i: (i, 0)),
            pl.BlockSpec((E, D), lambda i: (0, 0)),
            pl.BlockSpec((C, E), lambda i: (0, 0)),
        ],
        out_specs=pl.BlockSpec((BLK, C), lambda i: (i, 0)),
        out_shape=jax.ShapeDtypeStruct((B, C), jnp.float32),
    )(sums, W0, W_cls)


def kernel(features, W0, W_cls, nodes, neigh_idx):
    samp = jnp.concatenate([nodes[:, None], neigh_idx], axis=1)
    samp = jnp.pad(samp, ((0, BPAD - B), (0, 0)))
    idx_grp = samp.reshape(NW, NSTEPS, IPS)
    idx_grp = jnp.pad(idx_grp, ((0, 0), (0, 0), (0, IDXW - IPS)))
    sums = _sc_gather_sum(features, idx_grp)[:B]
    return _tc_head(sums, W0, W_cls)


# speedup vs baseline: 1.5319x; 1.5319x over previous
import functools

import numpy as np

import jax
import jax.numpy as jnp
from jax import lax
from jax.experimental import pallas as pl
from jax.experimental.pallas import tpu as pltpu
from jax.experimental.pallas import tpu_sc as plsc

B = 50000
D = 128
E = 128
C = 16
S1 = 11

NC, NS = 2, 16
NW = NC * NS
RPS = 8
IPS = RPS * S1
IDXW = 96
NSTEPS = 196
BPW = NSTEPS * RPS
BPAD = NW * BPW

_P = np.arange(64)
PERM = np.concatenate([32 * (_P // 16) + 2 * (_P % 16),
                       32 * (_P // 16) + 2 * (_P % 16) + 1]).astype(np.int32)


def _sc_gather_sum(features, idx_grp):
    mesh = plsc.VectorSubcoreMesh(core_axis_name="c", subcore_axis_name="s")
    NBUF = 4
    NGRP = NSTEPS // NBUF

    @functools.partial(
        pl.kernel,
        mesh=mesh,
        compiler_params=pltpu.CompilerParams(needs_layout_passes=False, use_tc_tiling_on_sc=False),
        out_type=jax.ShapeDtypeStruct((BPAD, D), jnp.float32),
        scratch_types=[
            pltpu.VMEM((NSTEPS, IDXW), jnp.int32),
            pltpu.VMEM((NBUF, IDXW, D // 2), jnp.int32),
            pltpu.VMEM((NBUF, RPS, D), jnp.float32),
            pltpu.SemaphoreType.DMA((NBUF,)),
            pltpu.SemaphoreType.DMA((NBUF,)),
        ],
    )
    def k(idx_hbm, feat_hbm, out_hbm, idx_slab, rows, acc, gsem, osem):
        wid = lax.axis_index("s") * NC + lax.axis_index("c")
        pltpu.sync_copy(idx_hbm.at[wid], idx_slab)

        for b in range(NBUF):
            pltpu.async_copy(feat_hbm.at[idx_slab.at[b]], rows.at[b], gsem.at[b])

        HIMASK = jnp.int32(-65536)

        def group(p, carry):
            for b in range(NBUF):
                i = p * NBUF + b
                pltpu.make_async_copy(
                    feat_hbm.at[idx_slab.at[i]], rows.at[b], gsem.at[b]).wait()
                @pl.when(p > 0)
                def _():
                    pltpu.make_async_copy(
                        acc.at[b], out_hbm.at[pl.ds(0, RPS)], osem.at[b]).wait()
                for r in range(RPS):
                    for g in range(D // 32):
                        def _ld(row):
                            return rows[b, row, pl.ds(g * 16, 16)]
                        xi = _ld(r * S1)
                        ve = lax.bitcast_convert_type(xi << 16, jnp.float32)
                        vo = lax.bitcast_convert_type(xi & HIMASK, jnp.float32)
                        for j in range(1, S1):
                            xi = _ld(r * S1 + j)
                            ve = ve + lax.bitcast_convert_type(xi << 16, jnp.float32)
                            vo = vo + lax.bitcast_convert_type(xi & HIMASK, jnp.float32)
                        acc[b, r, pl.ds(g * 16, 16)] = ve
                        acc[b, r, pl.ds(64 + g * 16, 16)] = vo
                pltpu.async_copy(
                    acc.at[b], out_hbm.at[pl.ds(wid * BPW + i * RPS, RPS)],
                    osem.at[b])
                @pl.when(i + NBUF < NSTEPS)
                def _():
                    pltpu.async_copy(
                        feat_hbm.at[idx_slab.at[i + NBUF]], rows.at[b],
                        gsem.at[b])
            return carry

        lax.fori_loop(0, NGRP, group, 0)
        for b in range(NBUF):
            pltpu.make_async_copy(
                acc.at[b], out_hbm.at[pl.ds(0, RPS)], osem.at[b]).wait()

    return k(idx_grp, features)


BLK = 1000


def _tc_head(sums, W0, W_cls):
    def body(x_ref, w0_ref, wc_ref, o_ref):
        x = x_ref[...] * (1.0 / S1)
        h = lax.dot_general(x, w0_ref[...], (((1,), (1,)), ((), ())),
                            preferred_element_type=jnp.float32)
        h = jnp.maximum(h, 0.0)
        s = lax.dot_general(h, wc_ref[...], (((1,), (1,)), ((), ())),
                            preferred_element_type=jnp.float32)
        o_ref[...] = jax.nn.sigmoid(s)

    return pl.pallas_call(
        body,
        grid=(B // BLK,),
        in_specs=[
            pl.BlockSpec((BLK, D), lambda i: (i, 0)),
            pl.BlockSpec((E, D), lambda i: (0, 0)),
            pl.BlockSpec((C, E), lambda i: (0, 0)),
        ],
        out_specs=pl.BlockSpec((BLK, C), lambda i: (i, 0)),
        out_shape=jax.ShapeDtypeStruct((B, C), jnp.float32),
    )(sums, W0, W_cls)


def kernel(features, W0, W_cls, nodes, neigh_idx):
    samp = jnp.concatenate([nodes[:, None], neigh_idx], axis=1)
    samp = jnp.pad(samp, ((0, BPAD - B), (0, 0)))
    idx_grp = samp.reshape(NW, NSTEPS, IPS)
    idx_grp = jnp.pad(idx_grp, ((0, 0), (0, 0), (0, IDXW - IPS)))
    fb = features.astype(jnp.bfloat16).reshape(-1, D // 2, 2)
    fi = jax.lax.bitcast_convert_type(fb, jnp.int32)
    sums = _sc_gather_sum(fi, idx_grp)[:B]
    return _tc_head(sums, W0[:, PERM], W_cls)


# speedup vs baseline: 4.2153x; 2.7516x over previous
import functools

import numpy as np

import jax
import jax.numpy as jnp
from jax import lax
from jax.experimental import pallas as pl
from jax.experimental.pallas import tpu as pltpu
from jax.experimental.pallas import tpu_sc as plsc

B = 50000
D = 128
E = 128
C = 16
S1 = 11

NC, NS = 2, 16
NW = NC * NS
RPS = 8
IPS = RPS * S1
IDXW = IPS
NSTEPS = 196
BPW = NSTEPS * RPS
BPAD = NW * BPW

_P = np.arange(64)
PERM = np.concatenate([32 * (_P // 16) + 2 * (_P % 16),
                       32 * (_P // 16) + 2 * (_P % 16) + 1]).astype(np.int32)


def _sc_gather_sum(features, idx_grp):
    mesh = plsc.VectorSubcoreMesh(core_axis_name="c", subcore_axis_name="s")
    NBUF = 4
    NGRP = NSTEPS // NBUF

    @functools.partial(
        pl.kernel,
        mesh=mesh,
        compiler_params=pltpu.CompilerParams(needs_layout_passes=False, use_tc_tiling_on_sc=False),
        out_type=jax.ShapeDtypeStruct((BPAD, D), jnp.float32),
        scratch_types=[
            pltpu.VMEM((NSTEPS, IDXW), jnp.int32),
            pltpu.VMEM((NBUF, IDXW, D // 2), jnp.int32),
            pltpu.VMEM((NBUF, RPS, D), jnp.float32),
            pltpu.SemaphoreType.DMA((NBUF,)),
            pltpu.SemaphoreType.DMA((NBUF,)),
        ],
    )
    def k(idx_hbm, feat_hbm, out_hbm, idx_slab, rows, acc, gsem, osem):
        wid = lax.axis_index("s") * NC + lax.axis_index("c")
        pltpu.sync_copy(idx_hbm.at[wid], idx_slab)

        for b in range(NBUF):
            pltpu.async_copy(feat_hbm.at[idx_slab.at[b]], rows.at[b], gsem.at[b])

        HIMASK = jnp.int32(-65536)

        def group(p, carry):
            for b in range(NBUF):
                i = p * NBUF + b
                pltpu.make_async_copy(
                    feat_hbm.at[idx_slab.at[i]], rows.at[b], gsem.at[b]).wait()
                @pl.when(p > 0)
                def _():
                    pltpu.make_async_copy(
                        acc.at[b], out_hbm.at[pl.ds(0, RPS)], osem.at[b]).wait()
                for r in range(RPS):
                    for g in range(D // 32):
                        def _ld(row):
                            return rows[b, row, pl.ds(g * 16, 16)]
                        xi = _ld(r * S1)
                        ve = lax.bitcast_convert_type(xi << 16, jnp.float32)
                        vo = lax.bitcast_convert_type(xi & HIMASK, jnp.float32)
                        for j in range(1, S1):
                            xi = _ld(r * S1 + j)
                            ve = ve + lax.bitcast_convert_type(xi << 16, jnp.float32)
                            vo = vo + lax.bitcast_convert_type(xi & HIMASK, jnp.float32)
                        acc[b, r, pl.ds(g * 16, 16)] = ve
                        acc[b, r, pl.ds(64 + g * 16, 16)] = vo
                pltpu.async_copy(
                    acc.at[b], out_hbm.at[pl.ds(wid * BPW + i * RPS, RPS)],
                    osem.at[b])
                @pl.when(i + NBUF < NSTEPS)
                def _():
                    pltpu.async_copy(
                        feat_hbm.at[idx_slab.at[i + NBUF]], rows.at[b],
                        gsem.at[b])
            return carry

        lax.fori_loop(0, NGRP, group, 0)
        for b in range(NBUF):
            pltpu.make_async_copy(
                acc.at[b], out_hbm.at[pl.ds(0, RPS)], osem.at[b]).wait()

    return k(idx_grp, features)


BLK = 1000


def _tc_head(sums, W0, W_cls):
    def body(x_ref, w0_ref, wc_ref, o_ref):
        x = x_ref[...] * (1.0 / S1)
        h = lax.dot_general(x, w0_ref[...], (((1,), (1,)), ((), ())),
                            preferred_element_type=jnp.float32)
        h = jnp.maximum(h, 0.0)
        s = lax.dot_general(h, wc_ref[...], (((1,), (1,)), ((), ())),
                            preferred_element_type=jnp.float32)
        o_ref[...] = jax.nn.sigmoid(s)

    return pl.pallas_call(
        body,
        grid=(B // BLK,),
        in_specs=[
            pl.BlockSpec((BLK, D), lambda i: (i, 0)),
            pl.BlockSpec((E, D), lambda i: (0, 0)),
            pl.BlockSpec((C, E), lambda i: (0, 0)),
        ],
        out_specs=pl.BlockSpec((BLK, C), lambda i: (i, 0)),
        out_shape=jax.ShapeDtypeStruct((B, C), jnp.float32),
    )(sums, W0, W_cls)


def kernel(features, W0, W_cls, nodes, neigh_idx):
    samp = jnp.concatenate([nodes[:, None], neigh_idx], axis=1)
    samp = jnp.pad(samp, ((0, BPAD - B), (0, 0)))
    idx_grp = samp.reshape(NW, NSTEPS, IPS)
    fb = features.astype(jnp.bfloat16).reshape(-1, D // 2, 2)
    fi = jax.lax.bitcast_convert_type(fb, jnp.int32)
    sums = _sc_gather_sum(fi, idx_grp)
    return _tc_head(sums, W0[:, PERM], W_cls)


# speedup vs baseline: 7.9624x; 1.8889x over previous
import functools

import jax
import jax.numpy as jnp
from jax import lax
from jax.experimental import pallas as pl
from jax.experimental.pallas import tpu as pltpu
from jax.experimental.pallas import tpu_sc as plsc

B = 50000
D = 128
E = 128
C = 16
S1 = 11

NC, NS = 2, 16
NW = NC * NS
RPS = 8
IPS = RPS * S1
IDXW = IPS
NSTEPS = 196
BPW = NSTEPS * RPS
BPAD = NW * BPW

NROWS = 50000
HALF = NROWS // 2


def _tc_pack(features):
    def body(xa_ref, xb_ref, o_ref):
        def pack64(x):
            u = lax.bitcast_convert_type(x, jnp.uint32) + jnp.uint32(0x8000)
            w = (u[:, :64] >> 16) | (u[:, 64:] & jnp.uint32(0xFFFF0000))
            return lax.bitcast_convert_type(w, jnp.int32)
        o_ref[...] = jnp.concatenate(
            [pack64(xa_ref[...]), pack64(xb_ref[...])], axis=1)

    PBLK = 1000
    return pl.pallas_call(
        body,
        grid=(HALF // PBLK,),
        in_specs=[pl.BlockSpec((PBLK, D), lambda i: (i, 0)),
                  pl.BlockSpec((PBLK, D), lambda i: (i + HALF // PBLK, 0))],
        out_specs=pl.BlockSpec((PBLK, D), lambda i: (i, 0)),
        out_shape=jax.ShapeDtypeStruct((HALF, D), jnp.int32),
    )(features, features)


def _sc_gather_sum(features, idx_grp):
    mesh = plsc.VectorSubcoreMesh(core_axis_name="c", subcore_axis_name="s")
    NBUF = 4
    NGRP = NSTEPS // NBUF

    @functools.partial(
        pl.kernel,
        mesh=mesh,
        compiler_params=pltpu.CompilerParams(needs_layout_passes=False, use_tc_tiling_on_sc=False),
        out_type=jax.ShapeDtypeStruct((BPAD, D), jnp.float32),
        scratch_types=[
            pltpu.VMEM((NSTEPS * IDXW,), jnp.int32),
            pltpu.VMEM((NBUF, IDXW, D // 2), jnp.int32),
            pltpu.VMEM((NBUF, RPS, D), jnp.float32),
            pltpu.SemaphoreType.DMA((NBUF,)),
            pltpu.SemaphoreType.DMA((NBUF,)),
        ],
    )
    def k(idx_hbm, feat_hbm, out_hbm, idx_slab, rows, acc, gsem, osem):
        wid = lax.axis_index("s") * NC + lax.axis_index("c")
        pltpu.sync_copy(idx_hbm.at[pl.ds(wid * (NSTEPS * IDXW), NSTEPS * IDXW)],
                        idx_slab)

        for b in range(NBUF):
            pltpu.async_copy(feat_hbm.at[idx_slab.at[pl.ds(b * IDXW, IDXW)]], rows.at[b], gsem.at[b])

        HIMASK = jnp.int32(-65536)

        def group(p, carry):
            for b in range(NBUF):
                i = p * NBUF + b
                pltpu.make_async_copy(
                    feat_hbm.at[idx_slab.at[pl.ds(i * IDXW, IDXW)]], rows.at[b], gsem.at[b]).wait()
                @pl.when(p > 0)
                def _():
                    pltpu.make_async_copy(
                        acc.at[b], out_hbm.at[pl.ds(0, RPS)], osem.at[b]).wait()
                for r in range(RPS):
                    for g in range(D // 32):
                        def _ld(row):
                            return rows[b, row, pl.ds(g * 16, 16)]
                        xi = _ld(r * S1)
                        ve = lax.bitcast_convert_type(xi << 16, jnp.float32)
                        vo = lax.bitcast_convert_type(xi & HIMASK, jnp.float32)
                        for j in range(1, S1):
                            xi = _ld(r * S1 + j)
                            ve = ve + lax.bitcast_convert_type(xi << 16, jnp.float32)
                            vo = vo + lax.bitcast_convert_type(xi & HIMASK, jnp.float32)
                        acc[b, r, pl.ds(g * 16, 16)] = ve
                        acc[b, r, pl.ds(64 + g * 16, 16)] = vo
                pltpu.async_copy(
                    acc.at[b], out_hbm.at[pl.ds(wid * BPW + i * RPS, RPS)],
                    osem.at[b])
                @pl.when(i + NBUF < NSTEPS)
                def _():
                    pltpu.async_copy(
                        feat_hbm.at[idx_slab.at[pl.ds((i + NBUF) * IDXW, IDXW)]], rows.at[b],
                        gsem.at[b])
            return carry

        lax.fori_loop(0, NGRP, group, 0)
        for b in range(NBUF):
            pltpu.make_async_copy(
                acc.at[b], out_hbm.at[pl.ds(0, RPS)], osem.at[b]).wait()

    return k(idx_grp, features)


BLK = 2000


def _tc_head(sums, W0, W_cls):
    def body(x_ref, w0_ref, wc_ref, o_ref):
        x = x_ref[...] * (1.0 / S1)
        h = lax.dot_general(x, w0_ref[...], (((1,), (1,)), ((), ())),
                            preferred_element_type=jnp.float32)
        h = jnp.maximum(h, 0.0)
        s = lax.dot_general(h, wc_ref[...], (((1,), (1,)), ((), ())),
                            preferred_element_type=jnp.float32)
        o_ref[...] = jax.nn.sigmoid(s)

    return pl.pallas_call(
        body,
        grid=(B // BLK,),
        in_specs=[
            pl.BlockSpec((BLK, D), lambda i: (i, 0)),
            pl.BlockSpec((E, D), lambda i: (0, 0)),
            pl.BlockSpec((C, E), lambda i: (0, 0)),
        ],
        out_specs=pl.BlockSpec((BLK, C), lambda i: (i, 0)),
        out_shape=jax.ShapeDtypeStruct((B, C), jnp.float32),
    )(sums, W0, W_cls)


def kernel(features, W0, W_cls, nodes, neigh_idx):
    samp = jnp.concatenate([nodes[:, None], neigh_idx], axis=1)
    filler = (jnp.arange(BPAD - B, dtype=jnp.int32)[:, None] * 11
              + jnp.arange(S1, dtype=jnp.int32)[None, :]) % jnp.int32(50000)
    samp = jnp.concatenate([samp, filler], axis=0)
    samp = jnp.where(samp < HALF, samp * 2, samp * 2 - (NROWS - 1))
    idx_flat = samp.reshape(-1)
    fi = _tc_pack(features).reshape(NROWS, D // 2)
    sums = _sc_gather_sum(fi, idx_flat)
    return _tc_head(sums, W0, W_cls)


# speedup vs baseline: 8.4814x; 1.0652x over previous
import functools

import jax
import jax.numpy as jnp
from jax import lax
from jax.experimental import pallas as pl
from jax.experimental.pallas import tpu as pltpu
from jax.experimental.pallas import tpu_sc as plsc

B = 50000
D = 128
E = 128
C = 16
S1 = 11

NC, NS = 2, 16
NW = NC * NS
RPS = 8
IPS = RPS * S1
IDXW = IPS
NSTEPS = 196
BPW = NSTEPS * RPS
BPAD = NW * BPW

NROWS = 50000
HALF = NROWS // 2


PBLK = 5000


def _tc_pack(features):
    def body(xa_ref, xb_ref, o_ref):
        def pack64(x):
            u = lax.bitcast_convert_type(x, jnp.uint32) + jnp.uint32(0x8000)
            w = (u[:, :64] >> 16) | (u[:, 64:] & jnp.uint32(0xFFFF0000))
            return lax.bitcast_convert_type(w, jnp.int32)
        w = jnp.concatenate(
            [pack64(xa_ref[...]), pack64(xb_ref[...])], axis=1)
        o_ref[...] = w.reshape(PBLK * D)

    return pl.pallas_call(
        body,
        grid=(HALF // PBLK,),
        in_specs=[pl.BlockSpec((PBLK, D), lambda i: (i, 0)),
                  pl.BlockSpec((PBLK, D), lambda i: (i + HALF // PBLK, 0))],
        out_specs=pl.BlockSpec((PBLK * D,), lambda i: (i,)),
        out_shape=jax.ShapeDtypeStruct((HALF * D,), jnp.int32),
    )(features, features)


def _sc_gather_sum(features, idx_grp):
    mesh = plsc.VectorSubcoreMesh(core_axis_name="c", subcore_axis_name="s")
    NBUF = 4
    NGRP = NSTEPS // NBUF

    @functools.partial(
        pl.kernel,
        mesh=mesh,
        compiler_params=pltpu.CompilerParams(needs_layout_passes=False, use_tc_tiling_on_sc=False),
        out_type=jax.ShapeDtypeStruct((BPAD, D), jnp.float32),
        scratch_types=[
            pltpu.VMEM((NSTEPS * IDXW,), jnp.int32),
            pltpu.VMEM((NBUF, IDXW, D // 2), jnp.int32),
            pltpu.VMEM((NBUF, RPS, D), jnp.float32),
            pltpu.SemaphoreType.DMA((NBUF,)),
            pltpu.SemaphoreType.DMA((NBUF,)),
        ],
    )
    def k(idx_hbm, feat_hbm, out_hbm, idx_slab, rows, acc, gsem, osem):
        wid = lax.axis_index("s") * NC + lax.axis_index("c")
        pltpu.sync_copy(idx_hbm.at[pl.ds(wid * (NSTEPS * IDXW), NSTEPS * IDXW)],
                        idx_slab)

        for b in range(NBUF):
            pltpu.async_copy(feat_hbm.at[idx_slab.at[pl.ds(b * IDXW, IDXW)]], rows.at[b], gsem.at[b])

        HIMASK = jnp.int32(-65536)

        def group(p, carry):
            for b in range(NBUF):
                i = p * NBUF + b
                pltpu.make_async_copy(
                    feat_hbm.at[idx_slab.at[pl.ds(i * IDXW, IDXW)]], rows.at[b], gsem.at[b]).wait()
                @pl.when(p > 0)
                def _():
                    pltpu.make_async_copy(
                        acc.at[b], out_hbm.at[pl.ds(0, RPS)], osem.at[b]).wait()
                for r in range(RPS):
                    for g in range(D // 32):
                        def _ld(row):
                            return rows[b, row, pl.ds(g * 16, 16)]
                        xi = _ld(r * S1)
                        ve = lax.bitcast_convert_type(xi << 16, jnp.float32)
                        vo = lax.bitcast_convert_type(xi & HIMASK, jnp.float32)
                        for j in range(1, S1):
                            xi = _ld(r * S1 + j)
                            ve = ve + lax.bitcast_convert_type(xi << 16, jnp.float32)
                            vo = vo + lax.bitcast_convert_type(xi & HIMASK, jnp.float32)
                        acc[b, r, pl.ds(g * 16, 16)] = ve
                        acc[b, r, pl.ds(64 + g * 16, 16)] = vo
                pltpu.async_copy(
                    acc.at[b], out_hbm.at[pl.ds(wid * BPW + i * RPS, RPS)],
                    osem.at[b])
                @pl.when(i + NBUF < NSTEPS)
                def _():
                    pltpu.async_copy(
                        feat_hbm.at[idx_slab.at[pl.ds((i + NBUF) * IDXW, IDXW)]], rows.at[b],
                        gsem.at[b])
            return carry

        lax.fori_loop(0, NGRP, group, 0)
        for b in range(NBUF):
            pltpu.make_async_copy(
                acc.at[b], out_hbm.at[pl.ds(0, RPS)], osem.at[b]).wait()

    return k(idx_grp, features)


BLK = 5000


def _tc_head(sums, W0, W_cls):
    def body(x_ref, w0_ref, wc_ref, o_ref):
        x = x_ref[...] * (1.0 / S1)
        h = lax.dot_general(x, w0_ref[...], (((1,), (1,)), ((), ())),
                            preferred_element_type=jnp.float32)
        h = jnp.maximum(h, 0.0)
        s = lax.dot_general(h, wc_ref[...], (((1,), (1,)), ((), ())),
                            preferred_element_type=jnp.float32)
        o_ref[...] = jax.nn.sigmoid(s)

    return pl.pallas_call(
        body,
        grid=(B // BLK,),
        in_specs=[
            pl.BlockSpec((BLK, D), lambda i: (i, 0)),
            pl.BlockSpec((E, D), lambda i: (0, 0)),
            pl.BlockSpec((C, E), lambda i: (0, 0)),
        ],
        out_specs=pl.BlockSpec((BLK, C), lambda i: (i, 0)),
        out_shape=jax.ShapeDtypeStruct((B, C), jnp.float32),
    )(sums, W0, W_cls)


def kernel(features, W0, W_cls, nodes, neigh_idx):
    samp = jnp.concatenate([nodes[:, None], neigh_idx], axis=1)
    filler = (jnp.arange(BPAD - B, dtype=jnp.int32)[:, None] * 11
              + jnp.arange(S1, dtype=jnp.int32)[None, :]) % jnp.int32(50000)
    samp = jnp.concatenate([samp, filler], axis=0)
    samp = jnp.where(samp < HALF, samp * 2, samp * 2 - (NROWS - 1))
    idx_flat = samp.reshape(-1)
    fi = _tc_pack(features).reshape(NROWS, D // 2)
    sums = _sc_gather_sum(fi, idx_flat)
    return _tc_head(sums, W0, W_cls)


# speedup vs baseline: 8.8580x; 1.0444x over previous
import functools

import jax
import jax.numpy as jnp
from jax import lax
from jax.experimental import pallas as pl
from jax.experimental.pallas import tpu as pltpu
from jax.experimental.pallas import tpu_sc as plsc

B = 50000
D = 128
E = 128
C = 16
S1 = 11

NC, NS = 2, 16
NW = NC * NS
RPS = 8
IPS = RPS * S1
IDXW = IPS
NSTEPS = 196
BPW = NSTEPS * RPS
BPAD = NW * BPW

NROWS = 50000
HALF = NROWS // 2


PBLK = 5000


def _tc_pack(features):
    def body(xa_ref, xb_ref, o_ref):
        def pack64(x):
            u = lax.bitcast_convert_type(x, jnp.uint32) + jnp.uint32(0x8000)
            w = (u[:, :64] >> 16) | (u[:, 64:] & jnp.uint32(0xFFFF0000))
            return lax.bitcast_convert_type(w, jnp.int32)
        w = jnp.concatenate(
            [pack64(xa_ref[...]), pack64(xb_ref[...])], axis=1)
        o_ref[...] = w.reshape(PBLK * D)

    return pl.pallas_call(
        body,
        grid=(HALF // PBLK,),
        in_specs=[pl.BlockSpec((PBLK, D), lambda i: (i, 0)),
                  pl.BlockSpec((PBLK, D), lambda i: (i + HALF // PBLK, 0))],
        out_specs=pl.BlockSpec((PBLK * D,), lambda i: (i,)),
        out_shape=jax.ShapeDtypeStruct((HALF * D,), jnp.int32),
    )(features, features)


def _sc_gather_sum(features, idx_grp):
    mesh = plsc.VectorSubcoreMesh(core_axis_name="c", subcore_axis_name="s")
    NBUF = 7
    NGRP = NSTEPS // NBUF

    @functools.partial(
        pl.kernel,
        mesh=mesh,
        compiler_params=pltpu.CompilerParams(needs_layout_passes=False, use_tc_tiling_on_sc=False),
        out_type=jax.ShapeDtypeStruct((BPAD, D), jnp.float32),
        scratch_types=[
            pltpu.VMEM((NSTEPS * IDXW,), jnp.int32),
            pltpu.VMEM((NBUF, IDXW, D // 2), jnp.int32),
            pltpu.VMEM((NBUF, RPS, D), jnp.float32),
            pltpu.SemaphoreType.DMA((NBUF,)),
            pltpu.SemaphoreType.DMA((NBUF,)),
        ],
    )
    def k(idx_hbm, feat_hbm, out_hbm, idx_slab, rows, acc, gsem, osem):
        wid = lax.axis_index("s") * NC + lax.axis_index("c")
        pltpu.sync_copy(idx_hbm.at[pl.ds(wid * (NSTEPS * IDXW), NSTEPS * IDXW)],
                        idx_slab)

        for b in range(NBUF):
            pltpu.async_copy(feat_hbm.at[idx_slab.at[pl.ds(b * IDXW, IDXW)]], rows.at[b], gsem.at[b])

        def group(p, carry):
            for b in range(NBUF):
                i = p * NBUF + b
                pltpu.make_async_copy(
                    feat_hbm.at[idx_slab.at[pl.ds(i * IDXW, IDXW)]], rows.at[b], gsem.at[b]).wait()
                @pl.when(p > 0)
                def _():
                    pltpu.make_async_copy(
                        acc.at[b], out_hbm.at[pl.ds(0, RPS)], osem.at[b]).wait()

                def acc_row(r, c):
                    for g in range(D // 32):
                        def _ld(j):
                            return rows[b, r * S1 + j, pl.ds(g * 16, 16)]
                        xi = _ld(0)
                        ve = lax.bitcast_convert_type(xi << 16, jnp.float32)
                        vo = lax.bitcast_convert_type(xi, jnp.float32)
                        for j in range(1, S1):
                            xi = _ld(j)
                            ve = ve + lax.bitcast_convert_type(xi << 16, jnp.float32)
                            vo = vo + lax.bitcast_convert_type(xi, jnp.float32)
                        acc[b, r, pl.ds(g * 16, 16)] = ve
                        acc[b, r, pl.ds(64 + g * 16, 16)] = vo
                    return c

                lax.fori_loop(0, RPS, acc_row, 0)
                pltpu.async_copy(
                    acc.at[b], out_hbm.at[pl.ds(wid * BPW + i * RPS, RPS)],
                    osem.at[b])
                @pl.when(i + NBUF < NSTEPS)
                def _():
                    pltpu.async_copy(
                        feat_hbm.at[idx_slab.at[pl.ds((i + NBUF) * IDXW, IDXW)]], rows.at[b],
                        gsem.at[b])
            return carry

        lax.fori_loop(0, NGRP, group, 0)
        for b in range(NBUF):
            pltpu.make_async_copy(
                acc.at[b], out_hbm.at[pl.ds(0, RPS)], osem.at[b]).wait()

    return k(idx_grp, features)


BLK = 5000


def _tc_head(sums, W0, W_cls):
    def body(x_ref, w0_ref, wc_ref, o_ref):
        x = x_ref[...] * (1.0 / S1)
        h = lax.dot_general(x, w0_ref[...], (((1,), (1,)), ((), ())),
                            preferred_element_type=jnp.float32)
        h = jnp.maximum(h, 0.0)
        s = lax.dot_general(h, wc_ref[...], (((1,), (1,)), ((), ())),
                            preferred_element_type=jnp.float32)
        o_ref[...] = jax.nn.sigmoid(s)

    return pl.pallas_call(
        body,
        grid=(B // BLK,),
        in_specs=[
            pl.BlockSpec((BLK, D), lambda i: (i, 0)),
            pl.BlockSpec((E, D), lambda i: (0, 0)),
            pl.BlockSpec((C, E), lambda i: (0, 0)),
        ],
        out_specs=pl.BlockSpec((BLK, C), lambda i: (i, 0)),
        out_shape=jax.ShapeDtypeStruct((B, C), jnp.float32),
    )(sums, W0, W_cls)


def kernel(features, W0, W_cls, nodes, neigh_idx):
    samp = jnp.concatenate([nodes[:, None], neigh_idx], axis=1)
    filler = (jnp.arange(BPAD - B, dtype=jnp.int32)[:, None] * 11
              + jnp.arange(S1, dtype=jnp.int32)[None, :]) % jnp.int32(50000)
    samp = jnp.concatenate([samp, filler], axis=0)
    samp = jnp.where(samp < HALF, samp * 2, samp * 2 - (NROWS - 1))
    idx_flat = samp.reshape(-1)
    fi = _tc_pack(features).reshape(NROWS, D // 2)
    sums = _sc_gather_sum(fi, idx_flat)
    return _tc_head(sums, W0, W_cls)


# speedup vs baseline: 9.4702x; 1.0691x over previous
import functools

import jax
import jax.numpy as jnp
from jax import lax
from jax.experimental import pallas as pl
from jax.experimental.pallas import tpu as pltpu
from jax.experimental.pallas import tpu_sc as plsc

B = 50000
D = 128
E = 128
C = 16
S1 = 11

NC, NS = 2, 16
NW = NC * NS
RPS = 16
IPS = RPS * S1
IDXW = IPS
NSTEPS = 98
BPW = NSTEPS * RPS
BPAD = NW * BPW

NROWS = 50000
HALF = NROWS // 2


PBLK = 5000


def _tc_pack(features):
    def body(xa_ref, xb_ref, o_ref):
        def pack64(x):
            u = lax.bitcast_convert_type(x, jnp.uint32) + jnp.uint32(0x8000)
            w = (u[:, :64] >> 16) | (u[:, 64:] & jnp.uint32(0xFFFF0000))
            return lax.bitcast_convert_type(w, jnp.int32)
        w = jnp.concatenate(
            [pack64(xa_ref[...]), pack64(xb_ref[...])], axis=1)
        o_ref[...] = w.reshape(PBLK * D)

    return pl.pallas_call(
        body,
        grid=(HALF // PBLK,),
        in_specs=[pl.BlockSpec((PBLK, D), lambda i: (i, 0)),
                  pl.BlockSpec((PBLK, D), lambda i: (i + HALF // PBLK, 0))],
        out_specs=pl.BlockSpec((PBLK * D,), lambda i: (i,)),
        out_shape=jax.ShapeDtypeStruct((HALF * D,), jnp.int32),
    )(features, features)


def _sc_gather_sum(features, idx_grp):
    mesh = plsc.VectorSubcoreMesh(core_axis_name="c", subcore_axis_name="s")
    NBUF = 7
    NGRP = NSTEPS // NBUF

    @functools.partial(
        pl.kernel,
        mesh=mesh,
        compiler_params=pltpu.CompilerParams(needs_layout_passes=False, use_tc_tiling_on_sc=False),
        out_type=jax.ShapeDtypeStruct((BPAD, D), jnp.float32),
        scratch_types=[
            pltpu.VMEM((NSTEPS * IDXW,), jnp.int32),
            pltpu.VMEM((NBUF, IDXW, D // 2), jnp.int32),
            pltpu.VMEM((NBUF, RPS, D), jnp.float32),
            pltpu.SemaphoreType.DMA((NBUF,)),
            pltpu.SemaphoreType.DMA((NBUF,)),
        ],
    )
    def k(idx_hbm, feat_hbm, out_hbm, idx_slab, rows, acc, gsem, osem):
        wid = lax.axis_index("s") * NC + lax.axis_index("c")
        pltpu.sync_copy(idx_hbm.at[pl.ds(wid * (NSTEPS * IDXW), NSTEPS * IDXW)],
                        idx_slab)

        for b in range(NBUF):
            pltpu.async_copy(feat_hbm.at[idx_slab.at[pl.ds(b * IDXW, IDXW)]], rows.at[b], gsem.at[b])

        def group(p, carry):
            for b in range(NBUF):
                i = p * NBUF + b
                pltpu.make_async_copy(
                    feat_hbm.at[idx_slab.at[pl.ds(i * IDXW, IDXW)]], rows.at[b], gsem.at[b]).wait()
                @pl.when(p > 0)
                def _():
                    pltpu.make_async_copy(
                        acc.at[b], out_hbm.at[pl.ds(0, RPS)], osem.at[b]).wait()

                def acc_row(r, c):
                    for g in range(D // 32):
                        def _ld(j):
                            return rows[b, r * S1 + j, pl.ds(g * 16, 16)]
                        xi = _ld(0)
                        ve = lax.bitcast_convert_type(xi << 16, jnp.float32)
                        vo = lax.bitcast_convert_type(xi, jnp.float32)
                        for j in range(1, S1):
                            xi = _ld(j)
                            ve = ve + lax.bitcast_convert_type(xi << 16, jnp.float32)
                            vo = vo + lax.bitcast_convert_type(xi, jnp.float32)
                        acc[b, r, pl.ds(g * 16, 16)] = ve
                        acc[b, r, pl.ds(64 + g * 16, 16)] = vo
                    return c

                lax.fori_loop(0, RPS, acc_row, 0)
                pltpu.async_copy(
                    acc.at[b], out_hbm.at[pl.ds(wid * BPW + i * RPS, RPS)],
                    osem.at[b])
                @pl.when(i + NBUF < NSTEPS)
                def _():
                    pltpu.async_copy(
                        feat_hbm.at[idx_slab.at[pl.ds((i + NBUF) * IDXW, IDXW)]], rows.at[b],
                        gsem.at[b])
            return carry

        lax.fori_loop(0, NGRP, group, 0)
        for b in range(NBUF):
            pltpu.make_async_copy(
                acc.at[b], out_hbm.at[pl.ds(0, RPS)], osem.at[b]).wait()

    return k(idx_grp, features)


BLK = 5000


def _tc_head(sums, W0, W_cls):
    def body(x_ref, w0_ref, wc_ref, o_ref):
        x = x_ref[...] * (1.0 / S1)
        h = lax.dot_general(x, w0_ref[...], (((1,), (1,)), ((), ())),
                            preferred_element_type=jnp.float32)
        h = jnp.maximum(h, 0.0)
        s = lax.dot_general(h, wc_ref[...], (((1,), (1,)), ((), ())),
                            preferred_element_type=jnp.float32)
        o_ref[...] = jax.nn.sigmoid(s)

    return pl.pallas_call(
        body,
        grid=(B // BLK,),
        in_specs=[
            pl.BlockSpec((BLK, D), lambda i: (i, 0)),
            pl.BlockSpec((E, D), lambda i: (0, 0)),
            pl.BlockSpec((C, E), lambda i: (0, 0)),
        ],
        out_specs=pl.BlockSpec((BLK, C), lambda i: (i, 0)),
        out_shape=jax.ShapeDtypeStruct((B, C), jnp.float32),
    )(sums, W0, W_cls)


def kernel(features, W0, W_cls, nodes, neigh_idx):
    samp = jnp.concatenate([nodes[:, None], neigh_idx], axis=1)
    filler = (jnp.arange(BPAD - B, dtype=jnp.int32)[:, None] * 11
              + jnp.arange(S1, dtype=jnp.int32)[None, :]) % jnp.int32(50000)
    samp = jnp.concatenate([samp, filler], axis=0)
    samp = jnp.where(samp < HALF, samp * 2, samp * 2 - (NROWS - 1))
    idx_flat = samp.reshape(-1)
    fi = _tc_pack(features).reshape(NROWS, D // 2)
    sums = _sc_gather_sum(fi, idx_flat)
    return _tc_head(sums, W0, W_cls)


# speedup vs baseline: 9.5793x; 1.0115x over previous
import functools

import jax
import jax.numpy as jnp
from jax import lax
from jax.experimental import pallas as pl
from jax.experimental.pallas import tpu as pltpu
from jax.experimental.pallas import tpu_sc as plsc

B = 50000
D = 128
E = 128
C = 16
S1 = 11

NC, NS = 2, 16
NW = NC * NS
RPS = 24
IPS = RPS * S1
IDXW = IPS
NSTEPS = 66
BPW = NSTEPS * RPS
BPAD = NW * BPW

NROWS = 50000
HALF = NROWS // 2


PBLK = 5000


def _tc_pack(features):
    def body(xa_ref, xb_ref, o_ref):
        def pack64(x):
            u = lax.bitcast_convert_type(x, jnp.uint32) + jnp.uint32(0x8000)
            w = (u[:, :64] >> 16) | (u[:, 64:] & jnp.uint32(0xFFFF0000))
            return lax.bitcast_convert_type(w, jnp.int32)
        w = jnp.concatenate(
            [pack64(xa_ref[...]), pack64(xb_ref[...])], axis=1)
        o_ref[...] = w.reshape(PBLK * D)

    return pl.pallas_call(
        body,
        grid=(HALF // PBLK,),
        in_specs=[pl.BlockSpec((PBLK, D), lambda i: (i, 0)),
                  pl.BlockSpec((PBLK, D), lambda i: (i + HALF // PBLK, 0))],
        out_specs=pl.BlockSpec((PBLK * D,), lambda i: (i,)),
        out_shape=jax.ShapeDtypeStruct((HALF * D,), jnp.int32),
    )(features, features)


def _sc_gather_sum(features, idx_grp):
    mesh = plsc.VectorSubcoreMesh(core_axis_name="c", subcore_axis_name="s")
    NBUF = 3
    NGRP = NSTEPS // NBUF

    @functools.partial(
        pl.kernel,
        mesh=mesh,
        compiler_params=pltpu.CompilerParams(needs_layout_passes=False, use_tc_tiling_on_sc=False),
        out_type=jax.ShapeDtypeStruct((BPAD, D), jnp.float32),
        scratch_types=[
            pltpu.VMEM((NSTEPS * IDXW,), jnp.int32),
            pltpu.VMEM((NBUF, IDXW, D // 2), jnp.int32),
            pltpu.VMEM((NBUF, RPS, D), jnp.float32),
            pltpu.SemaphoreType.DMA((NBUF,)),
            pltpu.SemaphoreType.DMA((NBUF,)),
        ],
    )
    def k(idx_hbm, feat_hbm, out_hbm, idx_slab, rows, acc, gsem, osem):
        wid = lax.axis_index("s") * NC + lax.axis_index("c")
        pltpu.sync_copy(idx_hbm.at[pl.ds(wid * (NSTEPS * IDXW), NSTEPS * IDXW)],
                        idx_slab)

        for b in range(NBUF):
            pltpu.async_copy(feat_hbm.at[idx_slab.at[pl.ds(b * IDXW, IDXW)]], rows.at[b], gsem.at[b])

        def group(p, carry):
            for b in range(NBUF):
                i = p * NBUF + b
                pltpu.make_async_copy(
                    feat_hbm.at[idx_slab.at[pl.ds(i * IDXW, IDXW)]], rows.at[b], gsem.at[b]).wait()
                @pl.when(p > 0)
                def _():
                    pltpu.make_async_copy(
                        acc.at[b], out_hbm.at[pl.ds(0, RPS)], osem.at[b]).wait()

                def acc_row(r, c):
                    for g in range(D // 32):
                        def _ld(j):
                            return rows[b, r * S1 + j, pl.ds(g * 16, 16)]
                        xi = _ld(0)
                        ve = lax.bitcast_convert_type(xi << 16, jnp.float32)
                        vo = lax.bitcast_convert_type(xi, jnp.float32)
                        for j in range(1, S1):
                            xi = _ld(j)
                            ve = ve + lax.bitcast_convert_type(xi << 16, jnp.float32)
                            vo = vo + lax.bitcast_convert_type(xi, jnp.float32)
                        acc[b, r, pl.ds(g * 16, 16)] = ve
                        acc[b, r, pl.ds(64 + g * 16, 16)] = vo
                    return c

                lax.fori_loop(0, RPS, acc_row, 0)
                pltpu.async_copy(
                    acc.at[b], out_hbm.at[pl.ds(wid * BPW + i * RPS, RPS)],
                    osem.at[b])
                @pl.when(i + NBUF < NSTEPS)
                def _():
                    pltpu.async_copy(
                        feat_hbm.at[idx_slab.at[pl.ds((i + NBUF) * IDXW, IDXW)]], rows.at[b],
                        gsem.at[b])
            return carry

        lax.fori_loop(0, NGRP, group, 0)
        for b in range(NBUF):
            pltpu.make_async_copy(
                acc.at[b], out_hbm.at[pl.ds(0, RPS)], osem.at[b]).wait()

    return k(idx_grp, features)


BLK = 5000


def _tc_head(sums, W0, W_cls):
    def body(x_ref, w0_ref, wc_ref, o_ref):
        x = x_ref[...] * (1.0 / S1)
        h = lax.dot_general(x, w0_ref[...], (((1,), (1,)), ((), ())),
                            preferred_element_type=jnp.float32)
        h = jnp.maximum(h, 0.0)
        s = lax.dot_general(h, wc_ref[...], (((1,), (1,)), ((), ())),
                            preferred_element_type=jnp.float32)
        o_ref[...] = jax.nn.sigmoid(s)

    return pl.pallas_call(
        body,
        grid=(B // BLK,),
        in_specs=[
            pl.BlockSpec((BLK, D), lambda i: (i, 0)),
            pl.BlockSpec((E, D), lambda i: (0, 0)),
            pl.BlockSpec((C, E), lambda i: (0, 0)),
        ],
        out_specs=pl.BlockSpec((BLK, C), lambda i: (i, 0)),
        out_shape=jax.ShapeDtypeStruct((B, C), jnp.float32),
    )(sums, W0, W_cls)


def kernel(features, W0, W_cls, nodes, neigh_idx):
    samp = jnp.concatenate([nodes[:, None], neigh_idx], axis=1)
    filler = (jnp.arange(BPAD - B, dtype=jnp.int32)[:, None] * 11
              + jnp.arange(S1, dtype=jnp.int32)[None, :]) % jnp.int32(50000)
    samp = jnp.concatenate([samp, filler], axis=0)
    samp = jnp.where(samp < HALF, samp * 2, samp * 2 - (NROWS - 1))
    idx_flat = samp.reshape(-1)
    fi = _tc_pack(features).reshape(NROWS, D // 2)
    sums = _sc_gather_sum(fi, idx_flat)
    return _tc_head(sums, W0, W_cls)
